# baseline (speedup 1.0000x reference)
import functools

import jax
import jax.numpy as jnp
from jax import lax
from jax.experimental import pallas as pl
from jax.experimental.pallas import tpu as pltpu
from jax.experimental.pallas import tpu_sc as plsc

_B = 4096
_F = 26
_D = 64
_R = _B * _F
_NC = 2
_NS = 16
_NW = _NC * _NS
_BPW = _B // _NW
_BPC = 8
_CHUNK = _BPC * _F
_RPW = _R // _NW
_CPW = _BPW // _BPC

_mesh = plsc.VectorSubcoreMesh(core_axis_name="c", subcore_axis_name="s")

_V = 100000
_CB = 4096


def _transpose_body(tt_ref, out_ref):
    out_ref[:, 0:_D] = tt_ref[...].T


_transpose = pl.pallas_call(
    _transpose_body,
    grid=(_V // _CB + 1,),
    in_specs=[pl.BlockSpec((_D, _CB), lambda i: (0, i))],
    out_specs=pl.BlockSpec((_CB, 128), lambda i: (i, 0)),
    out_shape=jax.ShapeDtypeStruct((_V, 128), jnp.float32),
)


@functools.partial(
    pl.kernel,
    mesh=_mesh,
    out_type=jax.ShapeDtypeStruct((_B, _F * _D), jnp.float32),
    scratch_types=[
        pltpu.VMEM((_RPW,), jnp.int32),
        pltpu.VMEM((3, _CHUNK, 128), jnp.float32),
        pltpu.VMEM((3, _BPC, _F * _D), jnp.float32),
        pltpu.SemaphoreType.DMA,
        pltpu.SemaphoreType.DMA,
    ],
)
def _gather_rows(idx_hbm, table_hbm, out_hbm, idx_v, rows_v, img_v, gsem, wsem):
    wid = lax.axis_index("s") * _NC + lax.axis_index("c")
    rbase = pl.multiple_of(wid * _RPW, _RPW)
    bbase = pl.multiple_of(wid * _BPW, _BPW)
    pltpu.sync_copy(idx_hbm.at[pl.ds(rbase, _RPW)], idx_v)

    def g_desc(j, p):
        off = j * _CHUNK if isinstance(j, int) else pl.multiple_of(j * _CHUNK, 8)
        return pltpu.make_async_copy(
            table_hbm.at[idx_v.at[pl.ds(off, _CHUNK)]], rows_v.at[p], gsem)

    def w_desc(j, q):
        off = j * _BPC if isinstance(j, int) else pl.multiple_of(j * _BPC, _BPC)
        return pltpu.make_async_copy(
            img_v.at[q], out_hbm.at[pl.ds(bbase + off, _BPC)], wsem)

    def assemble(p, q):
        for b in range(_BPC):
            for f in range(_F):
                r = b * _F + f
                for u in range(_D // 16):
                    img_v[q, b, pl.ds(f * _D + u * 16, 16)] = (
                        rows_v[p, r, pl.ds(u * 16, 16)])

    g_desc(0, 0).start()
    g_desc(1, 1).start()

    def step(j, s):
        @pl.when(j + 2 < _CPW)
        def _():
            g_desc(j + 2, (s + 2) % 3).start()
        g_desc(j, s).wait()
        @pl.when(j >= 3)
        def _():
            w_desc(j - 3, s).wait()
        assemble(s, s)
        w_desc(j, s).start()

    def body(i, carry):
        for s in range(3):
            step(i * 3 + s, s)
        return carry

    lax.fori_loop(0, (_CPW - 1) // 3, body, 0)
    step(_CPW - 1, (_CPW - 1) % 3)
    w_desc(_CPW - 3, (_CPW - 3) % 3).wait()
    w_desc(_CPW - 2, (_CPW - 2) % 3).wait()
    w_desc(_CPW - 1, (_CPW - 1) % 3).wait()


def kernel(indices, table):
    idx_flat = indices.astype(jnp.int32).reshape(_R)
    t128 = _transpose(table.T)
    return _gather_rows(idx_flat, t128)


# speedup vs baseline: 1.1110x; 1.1110x over previous
import functools

import jax
import jax.numpy as jnp
from jax import lax
from jax.experimental import pallas as pl
from jax.experimental.pallas import tpu as pltpu
from jax.experimental.pallas import tpu_sc as plsc

_B = 4096
_F = 26
_D = 64
_R = _B * _F
_NC = 2
_NS = 16
_NW = _NC * _NS
_BPW = _B // _NW
_BPC = 8
_CHUNK = _BPC * _F
_RPW = _R // _NW
_CPW = _BPW // _BPC

_mesh = plsc.VectorSubcoreMesh(core_axis_name="c", subcore_axis_name="s")

_V = 100000
_CB = 8192


def _transpose_body(tt_ref, out_ref):
    out_ref[:, 0:_D] = tt_ref[...].T


_transpose = pl.pallas_call(
    _transpose_body,
    grid=(_V // _CB + 1,),
    in_specs=[pl.BlockSpec((_D, _CB), lambda i: (0, i))],
    out_specs=pl.BlockSpec((_CB, 128), lambda i: (i, 0)),
    out_shape=jax.ShapeDtypeStruct((_V, 128), jnp.float32),
)


@functools.partial(
    pl.kernel,
    mesh=_mesh,
    out_type=jax.ShapeDtypeStruct((_B, _F * _D), jnp.float32),
    scratch_types=[
        pltpu.VMEM((_RPW,), jnp.int32),
        pltpu.VMEM((2, _CHUNK, 128), jnp.float32),
        pltpu.VMEM((2, _BPC, _F * _D), jnp.float32),
        pltpu.SemaphoreType.DMA,
        pltpu.SemaphoreType.DMA,
    ],
)
def _gather_rows(idx_hbm, table_hbm, out_hbm, idx_v, rows_v, img_v, gsem, wsem):
    wid = lax.axis_index("s") * _NC + lax.axis_index("c")
    rbase = pl.multiple_of(wid * _RPW, _RPW)
    bbase = pl.multiple_of(wid * _BPW, _BPW)
    pltpu.sync_copy(idx_hbm.at[pl.ds(rbase, _RPW)], idx_v)

    def g_desc(j, p):
        off = j * _CHUNK if isinstance(j, int) else pl.multiple_of(j * _CHUNK, 8)
        return pltpu.make_async_copy(
            table_hbm.at[idx_v.at[pl.ds(off, _CHUNK)]], rows_v.at[p], gsem)

    def w_desc(j, q):
        off = j * _BPC if isinstance(j, int) else pl.multiple_of(j * _BPC, _BPC)
        return pltpu.make_async_copy(
            img_v.at[q], out_hbm.at[pl.ds(bbase + off, _BPC)], wsem)

    def assemble(p, q):
        for b in range(_BPC):
            for f in range(_F):
                r = b * _F + f
                for u in range(_D // 16):
                    img_v[q, b, pl.ds(f * _D + u * 16, 16)] = (
                        rows_v[p, r, pl.ds(u * 16, 16)])

    g_desc(0, 0).start()

    def body(i, carry):
        for s in range(2):
            j = i * 2 + s
            nxt = 1 - s
            @pl.when(j + 1 < _CPW)
            def _():
                g_desc(j + 1, nxt).start()
            g_desc(j, s).wait()
            @pl.when(j >= 2)
            def _():
                w_desc(j - 2, s).wait()
            assemble(s, s)
            w_desc(j, s).start()
        return carry

    lax.fori_loop(0, _CPW // 2, body, 0)
    w_desc(_CPW - 2, 0).wait()
    w_desc(_CPW - 1, 1).wait()


def kernel(indices, table):
    idx_flat = indices.astype(jnp.int32).reshape(_R)
    t128 = _transpose(table.T)
    return _gather_rows(idx_flat, t128)


# speedup vs baseline: 1.1207x; 1.0087x over previous
import functools

import jax
import jax.numpy as jnp
from jax import lax
from jax.experimental import pallas as pl
from jax.experimental.pallas import tpu as pltpu
from jax.experimental.pallas import tpu_sc as plsc

_B = 4096
_F = 26
_D = 64
_R = _B * _F
_NC = 2
_NS = 16
_NW = _NC * _NS
_BPW = _B // _NW
_BPC = 8
_CHUNK = _BPC * _F
_RPW = _R // _NW
_CPW = _BPW // _BPC

_mesh = plsc.VectorSubcoreMesh(core_axis_name="c", subcore_axis_name="s")

_V = 100000
_CB = 12800


def _transpose_body(tt_ref, out_ref):
    out_ref[:, 0:_D] = tt_ref[...].T


_transpose = pl.pallas_call(
    _transpose_body,
    grid=(_V // _CB + 1,),
    in_specs=[pl.BlockSpec((_D, _CB), lambda i: (0, i))],
    out_specs=pl.BlockSpec((_CB, 128), lambda i: (i, 0)),
    out_shape=jax.ShapeDtypeStruct((_V, 128), jnp.float32),
)


@functools.partial(
    pl.kernel,
    mesh=_mesh,
    out_type=jax.ShapeDtypeStruct((_B, _F * _D), jnp.float32),
    scratch_types=[
        pltpu.VMEM((_RPW,), jnp.int32),
        pltpu.VMEM((2, _CHUNK, 128), jnp.float32),
        pltpu.VMEM((2, _BPC, _F * _D), jnp.float32),
        pltpu.SemaphoreType.DMA,
        pltpu.SemaphoreType.DMA,
    ],
)
def _gather_rows(idx_hbm, table_hbm, out_hbm, idx_v, rows_v, img_v, gsem, wsem):
    wid = lax.axis_index("s") * _NC + lax.axis_index("c")
    rbase = pl.multiple_of(wid * _RPW, _RPW)
    bbase = pl.multiple_of(wid * _BPW, _BPW)
    pltpu.sync_copy(idx_hbm.at[pl.ds(rbase, _RPW)], idx_v)

    def g_desc(j, p):
        off = j * _CHUNK if isinstance(j, int) else pl.multiple_of(j * _CHUNK, 8)
        return pltpu.make_async_copy(
            table_hbm.at[idx_v.at[pl.ds(off, _CHUNK)]], rows_v.at[p], gsem)

    def w_desc(j, q):
        off = j * _BPC if isinstance(j, int) else pl.multiple_of(j * _BPC, _BPC)
        return pltpu.make_async_copy(
            img_v.at[q], out_hbm.at[pl.ds(bbase + off, _BPC)], wsem)

    def assemble(p, q):
        for b in range(_BPC):
            for f in range(_F):
                r = b * _F + f
                for u in range(_D // 16):
                    img_v[q, b, pl.ds(f * _D + u * 16, 16)] = (
                        rows_v[p, r, pl.ds(u * 16, 16)])

    g_desc(0, 0).start()

    def body(i, carry):
        for s in range(2):
            j = i * 2 + s
            nxt = 1 - s
            @pl.when(j + 1 < _CPW)
            def _():
                g_desc(j + 1, nxt).start()
            g_desc(j, s).wait()
            @pl.when(j >= 2)
            def _():
                w_desc(j - 2, s).wait()
            assemble(s, s)
            w_desc(j, s).start()
        return carry

    lax.fori_loop(0, _CPW // 2, body, 0)
    w_desc(_CPW - 2, 0).wait()
    w_desc(_CPW - 1, 1).wait()


def kernel(indices, table):
    idx_flat = indices.astype(jnp.int32).reshape(_R)
    t128 = _transpose(table.T)
    return _gather_rows(idx_flat, t128)


# speedup vs baseline: 1.1389x; 1.0163x over previous
import functools

import jax
import jax.numpy as jnp
from jax import lax
from jax.experimental import pallas as pl
from jax.experimental.pallas import tpu as pltpu
from jax.experimental.pallas import tpu_sc as plsc

_B = 4096
_F = 26
_D = 64
_R = _B * _F
_NC = 2
_NS = 16
_NW = _NC * _NS
_BPW = _B // _NW
_BPC = 8
_CHUNK = _BPC * _F
_RPW = _R // _NW
_CPW = _BPW // _BPC

_mesh = plsc.VectorSubcoreMesh(core_axis_name="c", subcore_axis_name="s")

_V = 100000
_CB = 25088


def _transpose_body(tt_ref, out_ref):
    out_ref[:, 0:_D] = tt_ref[...].T


_transpose = pl.pallas_call(
    _transpose_body,
    grid=(_V // _CB + 1,),
    in_specs=[pl.BlockSpec((_D, _CB), lambda i: (0, i))],
    out_specs=pl.BlockSpec((_CB, 128), lambda i: (i, 0)),
    out_shape=jax.ShapeDtypeStruct((_V, 128), jnp.float32),
)


@functools.partial(
    pl.kernel,
    mesh=_mesh,
    out_type=jax.ShapeDtypeStruct((_B, _F * _D), jnp.float32),
    scratch_types=[
        pltpu.VMEM((_RPW,), jnp.int32),
        pltpu.VMEM((2, _CHUNK, 128), jnp.float32),
        pltpu.VMEM((2, _BPC, _F * _D), jnp.float32),
        pltpu.SemaphoreType.DMA,
        pltpu.SemaphoreType.DMA,
    ],
)
def _gather_rows(idx_hbm, table_hbm, out_hbm, idx_v, rows_v, img_v, gsem, wsem):
    wid = lax.axis_index("s") * _NC + lax.axis_index("c")
    rbase = pl.multiple_of(wid * _RPW, _RPW)
    bbase = pl.multiple_of(wid * _BPW, _BPW)
    pltpu.sync_copy(idx_hbm.at[pl.ds(rbase, _RPW)], idx_v)

    def g_desc(j, p):
        off = j * _CHUNK if isinstance(j, int) else pl.multiple_of(j * _CHUNK, 8)
        return pltpu.make_async_copy(
            table_hbm.at[idx_v.at[pl.ds(off, _CHUNK)]], rows_v.at[p], gsem)

    def w_desc(j, q):
        off = j * _BPC if isinstance(j, int) else pl.multiple_of(j * _BPC, _BPC)
        return pltpu.make_async_copy(
            img_v.at[q], out_hbm.at[pl.ds(bbase + off, _BPC)], wsem)

    def assemble(p, q):
        for b in range(_BPC):
            for f in range(_F):
                r = b * _F + f
                for u in range(_D // 16):
                    img_v[q, b, pl.ds(f * _D + u * 16, 16)] = (
                        rows_v[p, r, pl.ds(u * 16, 16)])

    g_desc(0, 0).start()

    def body(i, carry):
        for s in range(2):
            j = i * 2 + s
            nxt = 1 - s
            @pl.when(j + 1 < _CPW)
            def _():
                g_desc(j + 1, nxt).start()
            g_desc(j, s).wait()
            @pl.when(j >= 2)
            def _():
                w_desc(j - 2, s).wait()
            assemble(s, s)
            w_desc(j, s).start()
        return carry

    lax.fori_loop(0, _CPW // 2, body, 0)
    w_desc(_CPW - 2, 0).wait()
    w_desc(_CPW - 1, 1).wait()


def kernel(indices, table):
    idx_flat = indices.astype(jnp.int32).reshape(_R)
    t128 = _transpose(table.T)
    return _gather_rows(idx_flat, t128)


# speedup vs baseline: 1.1611x; 1.0195x over previous
import functools

import jax
import jax.numpy as jnp
from jax import lax
from jax.experimental import pallas as pl
from jax.experimental.pallas import tpu as pltpu
from jax.experimental.pallas import tpu_sc as plsc

_B = 4096
_F = 26
_D = 64
_R = _B * _F
_NC = 2
_NS = 16
_NW = _NC * _NS
_BPW = _B // _NW
_NP = _F // 2

_mesh = plsc.VectorSubcoreMesh(core_axis_name="c", subcore_axis_name="s")

_V = 100000
_CB = 25088


def _transpose_body(tt_ref, out_ref):
    out_ref[:, 0:_D] = tt_ref[...].T


_transpose = pl.pallas_call(
    _transpose_body,
    grid=(_V // _CB + 1,),
    in_specs=[pl.BlockSpec((_D, _CB), lambda i: (0, i))],
    out_specs=pl.BlockSpec((_CB, 128), lambda i: (i, 0)),
    out_shape=jax.ShapeDtypeStruct((_V, 128), jnp.float32),
)


@functools.partial(
    pl.kernel,
    mesh=_mesh,
    out_type=jax.ShapeDtypeStruct((_B, _F * _D), jnp.float32),
    scratch_types=[
        pltpu.VMEM((_F, _BPW), jnp.int32),
        pltpu.VMEM((4, _BPW, 128), jnp.float32),
        pltpu.VMEM((2, _BPW, 128), jnp.float32),
        pltpu.SemaphoreType.DMA,
        pltpu.SemaphoreType.DMA,
    ],
)
def _gather_rows(idx_hbm, table_hbm, out_hbm, idx_v, rows_v, img_v, gsem, wsem):
    wid = lax.axis_index("s") * _NC + lax.axis_index("c")
    bbase = pl.multiple_of(wid * _BPW, _BPW)
    pltpu.sync_copy(idx_hbm.at[:, pl.ds(bbase, _BPW)], idx_v)

    def g_desc(f, s):
        return pltpu.make_async_copy(
            table_hbm.at[idx_v.at[f]], rows_v.at[s], gsem)

    def w_desc(p, q):
        return pltpu.make_async_copy(
            img_v.at[q],
            out_hbm.at[pl.ds(bbase, _BPW), pl.ds(p * 128, 128)], wsem)

    def assemble(sa, sb, q):
        for b in range(_BPW):
            for u in range(_D // 16):
                img_v[q, b, pl.ds(u * 16, 16)] = rows_v[sa, b, pl.ds(u * 16, 16)]
                img_v[q, b, pl.ds(_D + u * 16, 16)] = (
                    rows_v[sb, b, pl.ds(u * 16, 16)])

    def step(p, sa, sb, q):
        @pl.when(p + 1 < _NP)
        def _():
            g_desc(2 * p + 2, (sa + 2) % 4).start()
            g_desc(2 * p + 3, (sb + 2) % 4).start()
        g_desc(2 * p, sa).wait()
        g_desc(2 * p + 1, sb).wait()
        @pl.when(p >= 2)
        def _():
            w_desc(p - 2, q).wait()
        assemble(sa, sb, q)
        w_desc(p, q).start()

    g_desc(0, 0).start()
    g_desc(1, 1).start()

    def body(i, carry):
        for k in range(2):
            step(i * 2 + k, 2 * k, 2 * k + 1, k)
        return carry

    lax.fori_loop(0, _NP // 2, body, 0)
    step(_NP - 1, 0, 1, 0)
    w_desc(_NP - 2, 1).wait()
    w_desc(_NP - 1, 0).wait()


def kernel(indices, table):
    t128 = _transpose(table.T)
    return _gather_rows(indices.astype(jnp.int32).T, t128)
